# baseline (speedup 1.0000x reference)
import jax
import jax.numpy as jnp
from jax import lax
from jax.experimental import pallas as pl
from jax.experimental.pallas import tpu as pltpu
from jax.experimental.pallas import tpu_sc as plsc

_N = 10000
_D = 128
_B = 64
_ROW = 640
_ACC = _B * _ROW
_NW = 32
_CHUNK = 320
_TAIL = _N - (_NW - 1) * _CHUNK
_G = 16


def _sc_moments(graph_hbm, idx_hbm, part_hbm, chunk_v, idx_v, acc_v):
    wid = lax.axis_index("s") * 2 + lax.axis_index("c")
    base = wid * _CHUNK

    zeros16 = jnp.zeros((16,), jnp.float32)

    def zbody(j, carry):
        for k in range(8):
            acc_v[pl.ds(j * 128 + k * 16, 16)] = zeros16
        return carry

    lax.fori_loop(0, _ACC // 128, zbody, 0)

    @pl.when(wid < _NW - 1)
    def _():
        pltpu.sync_copy(graph_hbm.at[pl.ds(base * _D, _CHUNK * _D)], chunk_v)
        pltpu.sync_copy(idx_hbm.at[pl.ds(base, _CHUNK)],
                        idx_v.at[pl.ds(0, _CHUNK)])

    @pl.when(wid == _NW - 1)
    def _():
        pltpu.sync_copy(graph_hbm.at[pl.ds(base * _D, _TAIL * _D)],
                        chunk_v.at[pl.ds(0, _TAIL * _D)])
        pltpu.sync_copy(idx_hbm.at[pl.ds(base, _TAIL)],
                        idx_v.at[pl.ds(0, _TAIL)])

    ngroups = jnp.where(wid == _NW - 1, _TAIL // _G, _CHUNK // _G)
    ones16 = jnp.ones((16,), jnp.float32)

    def body(gi, carry):
        i0 = gi * _G
        idxv = idx_v[pl.ds(i0, _G)]
        s_first = idxv[0]
        s_last = idxv[_G - 1]
        ib = i0 * _D

        @pl.when(s_first == s_last)
        def _():
            rb = s_first * _ROW
            for g in range(_D // 16):
                a1 = [zeros16] * 4
                a2 = [zeros16] * 4
                a3 = [zeros16] * 4
                a4 = [zeros16] * 4
                for j in range(_G):
                    w = j % 4
                    x = chunk_v[pl.ds(ib + j * _D + g * 16, 16)]
                    x2 = x * x
                    a1[w] = a1[w] + x
                    a2[w] = a2[w] + x2
                    a3[w] = a3[w] + x2 * x
                    a4[w] = a4[w] + x2 * x2
                s1v = (a1[0] + a1[1]) + (a1[2] + a1[3])
                s2v = (a2[0] + a2[1]) + (a2[2] + a2[3])
                s3v = (a3[0] + a3[1]) + (a3[2] + a3[3])
                s4v = (a4[0] + a4[1]) + (a4[2] + a4[3])
                plsc.addupdate(acc_v.at[pl.ds(rb + g * 16, 16)], s1v)
                plsc.addupdate(acc_v.at[pl.ds(rb + _D + g * 16, 16)], s2v)
                plsc.addupdate(acc_v.at[pl.ds(rb + 2 * _D + g * 16, 16)], s3v)
                plsc.addupdate(acc_v.at[pl.ds(rb + 3 * _D + g * 16, 16)], s4v)
            plsc.addupdate(acc_v.at[pl.ds(rb + 4 * _D, 16)], ones16 * float(_G))

        @pl.when(s_first != s_last)
        def _():
            for j in range(_G):
                s = idxv[j]
                rb = s * _ROW
                jb = ib + j * _D
                for g in range(_D // 16):
                    x = chunk_v[pl.ds(jb + g * 16, 16)]
                    x2 = x * x
                    plsc.addupdate(acc_v.at[pl.ds(rb + g * 16, 16)], x)
                    plsc.addupdate(acc_v.at[pl.ds(rb + _D + g * 16, 16)], x2)
                    plsc.addupdate(acc_v.at[pl.ds(rb + 2 * _D + g * 16, 16)],
                                   x2 * x)
                    plsc.addupdate(acc_v.at[pl.ds(rb + 3 * _D + g * 16, 16)],
                                   x2 * x2)
                plsc.addupdate(acc_v.at[pl.ds(rb + 4 * _D, 16)], ones16)

        return carry

    lax.fori_loop(0, ngroups, body, 0)

    pltpu.sync_copy(acc_v, part_hbm.at[pl.ds(wid * _ACC, _ACC)])


def _finalize(part_ref, out_ref):
    tot = jnp.sum(part_ref[...], axis=0)
    s1 = tot[:, 0:_D]
    s2 = tot[:, _D:2 * _D]
    s3 = tot[:, 2 * _D:3 * _D]
    s4 = tot[:, 3 * _D:4 * _D]
    cnt = tot[:, 4 * _D:4 * _D + 1]
    m = s1 / cnt
    m2 = s2 / cnt
    m3 = s3 / cnt
    m4 = s4 / cnt
    mm = m * m
    var = m2 - mm
    skew = m3 - 3.0 * m * m2 + 2.0 * m * mm
    kurt = m4 - 4.0 * m * m3 + 6.0 * mm * m2 - 3.0 * mm * mm - 3.0
    inf_val = 1000000000000000.0
    skew = jnp.where(skew > inf_val, 0.0, skew)
    skew = jnp.where(jnp.isnan(skew), 0.0, skew)
    kurt = jnp.where(kurt > inf_val, -3.0, kurt)
    kurt = jnp.where(jnp.isnan(kurt), -3.0, kurt)
    out_ref[...] = jnp.concatenate([m, var, skew, kurt], axis=1)


def kernel(graph, batch_indices):
    graph1d = jnp.reshape(graph, (-1,))
    part = pl.kernel(
        _sc_moments,
        out_type=jax.ShapeDtypeStruct((_NW * _ACC,), jnp.float32),
        mesh=plsc.VectorSubcoreMesh(core_axis_name="c", subcore_axis_name="s"),
        scratch_types=[
            pltpu.VMEM((_CHUNK * _D,), jnp.float32),
            pltpu.VMEM((_CHUNK + 16,), jnp.int32),
            pltpu.VMEM((_ACC,), jnp.float32),
        ],
    )(graph1d, batch_indices)
    part3 = jnp.reshape(part, (_NW, _B, _ROW))
    return pl.pallas_call(
        _finalize,
        out_shape=jax.ShapeDtypeStruct((_B, 4 * _D), jnp.float32),
    )(part3)


# speedup vs baseline: 1.2955x; 1.2955x over previous
import jax
import jax.numpy as jnp
from jax import lax
from jax.experimental import pallas as pl
from jax.experimental.pallas import tpu as pltpu
from jax.experimental.pallas import tpu_sc as plsc

_N = 10000
_D = 128
_B = 64
_ROW = 640
_NW = 32
_TC_ROWS = 4880
_CHUNK = (_N - _TC_ROWS) // _NW
_G = 16
_NGROUPS = _CHUNK // _G
_AROWS = _B + _G


def _sc_moments(graph_hbm, idx_hbm, part_hbm, chunk_v, idx_v, acc_v,
                zstg_v, widx_v, fidx_v, shared):
    cid = lax.axis_index("c")
    sid = lax.axis_index("s")
    wid = sid * 2 + cid
    base = _TC_ROWS + wid * _CHUNK

    zeros16 = jnp.zeros((16,), jnp.float32)
    iota16 = lax.iota(jnp.int32, 16)

    for k in range(5):
        for j in range(_ROW // 16):
            zstg_v[k, pl.ds(j * 16, 16)] = zeros16
    pltpu.sync_copy(zstg_v, shared.at[pl.ds(5 * sid, 5)])

    pltpu.sync_copy(graph_hbm.at[pl.ds(base * _D, _CHUNK * _D)], chunk_v)
    pltpu.sync_copy(idx_hbm.at[pl.ds(base, _CHUNK)],
                    idx_v.at[pl.ds(0, _CHUNK)])

    smin = idx_v[pl.ds(0, 16)][0]
    smax = idx_v[pl.ds(_CHUNK - 16, 16)][15]
    narrow = (smax - smin) < _G

    @pl.when(narrow)
    def _():
        def zw(r, carry):
            for j in range(_ROW // 16):
                acc_v[r, pl.ds(j * 16, 16)] = zeros16
            return carry
        lax.fori_loop(smin, smin + _G, zw, 0)

    @pl.when(jnp.logical_not(narrow))
    def _():
        def zf(r, carry):
            for j in range(_ROW // 16):
                acc_v[r, pl.ds(j * 16, 16)] = zeros16
            return carry
        lax.fori_loop(0, _B, zf, 0)

    ones16 = jnp.ones((16,), jnp.float32)

    def body(gi, carry):
        i0 = gi * _G
        idxv = idx_v[pl.ds(i0, _G)]
        s_first = idxv[0]
        s_last = idxv[_G - 1]
        ib = i0 * _D

        @pl.when(s_first == s_last)
        def _():
            for g in range(_D // 16):
                a1 = [zeros16] * 4
                a2 = [zeros16] * 4
                a3 = [zeros16] * 4
                a4 = [zeros16] * 4
                for j in range(_G):
                    w = j % 4
                    x = chunk_v[pl.ds(ib + j * _D + g * 16, 16)]
                    x2 = x * x
                    a1[w] = a1[w] + x
                    a2[w] = a2[w] + x2
                    a3[w] = a3[w] + x2 * x
                    a4[w] = a4[w] + x2 * x2
                s1v = (a1[0] + a1[1]) + (a1[2] + a1[3])
                s2v = (a2[0] + a2[1]) + (a2[2] + a2[3])
                s3v = (a3[0] + a3[1]) + (a3[2] + a3[3])
                s4v = (a4[0] + a4[1]) + (a4[2] + a4[3])
                plsc.addupdate(acc_v.at[s_first, pl.ds(g * 16, 16)], s1v)
                plsc.addupdate(acc_v.at[s_first, pl.ds(_D + g * 16, 16)], s2v)
                plsc.addupdate(acc_v.at[s_first, pl.ds(2 * _D + g * 16, 16)],
                               s3v)
                plsc.addupdate(acc_v.at[s_first, pl.ds(3 * _D + g * 16, 16)],
                               s4v)
            plsc.addupdate(acc_v.at[s_first, pl.ds(4 * _D, 16)],
                           ones16 * float(_G))

        @pl.when(s_first != s_last)
        def _():
            for j in range(_G):
                s = idxv[j]
                jb = ib + j * _D
                for g in range(_D // 16):
                    x = chunk_v[pl.ds(jb + g * 16, 16)]
                    x2 = x * x
                    plsc.addupdate(acc_v.at[s, pl.ds(g * 16, 16)], x)
                    plsc.addupdate(acc_v.at[s, pl.ds(_D + g * 16, 16)], x2)
                    plsc.addupdate(acc_v.at[s, pl.ds(2 * _D + g * 16, 16)],
                                   x2 * x)
                    plsc.addupdate(acc_v.at[s, pl.ds(3 * _D + g * 16, 16)],
                                   x2 * x2)
                plsc.addupdate(acc_v.at[s, pl.ds(4 * _D, 16)], ones16)

        return carry

    lax.fori_loop(0, _NGROUPS, body, 0)

    plsc.subcore_barrier()

    @pl.when(narrow)
    def _():
        widx_v[pl.ds(0, 16)] = jnp.where(smin + iota16 <= smax,
                                         smin + iota16, _B)
        pltpu.sync_copy(acc_v.at[pl.ds(smin, _G)], shared.at[widx_v],
                        add=True)

    @pl.when(jnp.logical_not(narrow))
    def _():
        for k in range(_B // 16):
            fidx_v[pl.ds(k * 16, 16)] = iota16 + (k * 16)
        pltpu.sync_copy(acc_v.at[pl.ds(0, _B)], shared.at[fidx_v], add=True)

    plsc.subcore_barrier()

    pltpu.sync_copy(shared.at[pl.ds(4 * sid, 4)],
                    part_hbm.at[cid, pl.ds(4 * sid, 4)])


def _tc_moments(graph_ref, idx_ref, out_ref):
    x = graph_ref[...][:_TC_ROWS, :]
    ids = idx_ref[...][:, :_TC_ROWS]
    oh = (lax.broadcasted_iota(jnp.int32, (_B, _TC_ROWS), 0)
          == ids).astype(jnp.float32)
    x2 = x * x
    s1 = jnp.dot(oh, x, preferred_element_type=jnp.float32)
    s2 = jnp.dot(oh, x2, preferred_element_type=jnp.float32)
    s3 = jnp.dot(oh, x2 * x, preferred_element_type=jnp.float32)
    s4 = jnp.dot(oh, x2 * x2, preferred_element_type=jnp.float32)
    cnt = jnp.sum(oh, axis=1, keepdims=True)
    out_ref[...] = jnp.concatenate(
        [s1, s2, s3, s4, jnp.broadcast_to(cnt, (_B, 16)),
         jnp.zeros((_B, _ROW - 4 * _D - 16), jnp.float32)], axis=1)


def _finalize(part_ref, tcp_ref, out_ref):
    tot = jnp.sum(part_ref[...], axis=0) + tcp_ref[...]
    s1 = tot[:, 0:_D]
    s2 = tot[:, _D:2 * _D]
    s3 = tot[:, 2 * _D:3 * _D]
    s4 = tot[:, 3 * _D:4 * _D]
    cnt = tot[:, 4 * _D:4 * _D + 1]
    m = s1 / cnt
    m2 = s2 / cnt
    m3 = s3 / cnt
    m4 = s4 / cnt
    mm = m * m
    var = m2 - mm
    skew = m3 - 3.0 * m * m2 + 2.0 * m * mm
    kurt = m4 - 4.0 * m * m3 + 6.0 * mm * m2 - 3.0 * mm * mm - 3.0
    inf_val = 1000000000000000.0
    skew = jnp.where(skew > inf_val, 0.0, skew)
    skew = jnp.where(jnp.isnan(skew), 0.0, skew)
    kurt = jnp.where(kurt > inf_val, -3.0, kurt)
    kurt = jnp.where(jnp.isnan(kurt), -3.0, kurt)
    out_ref[...] = jnp.concatenate([m, var, skew, kurt], axis=1)


def kernel(graph, batch_indices):
    graph1d = jnp.reshape(graph, (-1,))
    part = pl.kernel(
        _sc_moments,
        out_type=jax.ShapeDtypeStruct((2, _B, _ROW), jnp.float32),
        mesh=plsc.VectorSubcoreMesh(core_axis_name="c", subcore_axis_name="s"),
        compiler_params=pltpu.CompilerParams(use_tc_tiling_on_sc=False),
        scratch_types=[
            pltpu.VMEM((_CHUNK * _D,), jnp.float32),
            pltpu.VMEM((_CHUNK + 16,), jnp.int32),
            pltpu.VMEM((_AROWS, _ROW), jnp.float32),
            pltpu.VMEM((5, _ROW), jnp.float32),
            pltpu.VMEM((16,), jnp.int32),
            pltpu.VMEM((_B,), jnp.int32),
            pltpu.VMEM_SHARED((5 * 16, _ROW), jnp.float32),
        ],
    )(graph1d, batch_indices)
    tc_part = pl.pallas_call(
        _tc_moments,
        out_shape=jax.ShapeDtypeStruct((_B, _ROW), jnp.float32),
    )(graph, jnp.reshape(batch_indices, (1, _N)))
    return pl.pallas_call(
        _finalize,
        out_shape=jax.ShapeDtypeStruct((_B, 4 * _D), jnp.float32),
    )(part, tc_part)


# speedup vs baseline: 1.3414x; 1.0354x over previous
import jax
import jax.numpy as jnp
from jax import lax
from jax.experimental import pallas as pl
from jax.experimental.pallas import tpu as pltpu
from jax.experimental.pallas import tpu_sc as plsc

_N = 10000
_D = 128
_B = 64
_ROW = 640
_NW = 32
_TC_ROWS = 4880
_CHUNK = (_N - _TC_ROWS) // _NW
_G = 16
_NGROUPS = _CHUNK // _G
_AROWS = _B + _G


def _sc_moments(graph_hbm, idx_hbm, part_hbm, chunk_v, idx_v, acc_v,
                zstg_v, widx_v, fidx_v, shared, gsem, isem):
    cid = lax.axis_index("c")
    sid = lax.axis_index("s")
    wid = sid * 2 + cid
    base = _TC_ROWS + wid * _CHUNK

    zeros16 = jnp.zeros((16,), jnp.float32)
    iota16 = lax.iota(jnp.int32, 16)

    gcopy = pltpu.async_copy(graph_hbm.at[pl.ds(base * _D, _CHUNK * _D)],
                             chunk_v, gsem)
    icopy = pltpu.async_copy(idx_hbm.at[pl.ds(base, _CHUNK)],
                             idx_v.at[pl.ds(0, _CHUNK)], isem)

    for k in range(5):
        for j in range(_ROW // 16):
            zstg_v[k, pl.ds(j * 16, 16)] = zeros16
    pltpu.sync_copy(zstg_v, shared.at[pl.ds(5 * sid, 5)])

    icopy.wait()
    smin = idx_v[pl.ds(0, 16)][0]
    smax = idx_v[pl.ds(_CHUNK - 16, 16)][15]
    narrow = (smax - smin) < _G

    @pl.when(narrow)
    def _():
        def zw(r, carry):
            for j in range(_ROW // 16):
                acc_v[r, pl.ds(j * 16, 16)] = zeros16
            return carry
        lax.fori_loop(smin, smin + _G, zw, 0)

    @pl.when(jnp.logical_not(narrow))
    def _():
        def zf(r, carry):
            for j in range(_ROW // 16):
                acc_v[r, pl.ds(j * 16, 16)] = zeros16
            return carry
        lax.fori_loop(0, _B, zf, 0)

    ones16 = jnp.ones((16,), jnp.float32)
    gcopy.wait()

    def body_fast(gi, carry):
        i0 = gi * _G
        idxv = idx_v[pl.ds(i0, _G)]
        s_first = idxv[0]
        s_last = idxv[_G - 1]
        ib = i0 * _D

        @pl.when(s_first == s_last)
        def _():
            for g in range(_D // 16):
                a1 = [zeros16] * 4
                a2 = [zeros16] * 4
                a3 = [zeros16] * 4
                a4 = [zeros16] * 4
                for j in range(_G):
                    w = j % 4
                    x = chunk_v[pl.ds(ib + j * _D + g * 16, 16)]
                    x2 = x * x
                    a1[w] = a1[w] + x
                    a2[w] = a2[w] + x2
                    a3[w] = a3[w] + x2 * x
                    a4[w] = a4[w] + x2 * x2
                s1v = (a1[0] + a1[1]) + (a1[2] + a1[3])
                s2v = (a2[0] + a2[1]) + (a2[2] + a2[3])
                s3v = (a3[0] + a3[1]) + (a3[2] + a3[3])
                s4v = (a4[0] + a4[1]) + (a4[2] + a4[3])
                plsc.addupdate(acc_v.at[s_first, pl.ds(g * 16, 16)], s1v)
                plsc.addupdate(acc_v.at[s_first, pl.ds(_D + g * 16, 16)], s2v)
                plsc.addupdate(acc_v.at[s_first, pl.ds(2 * _D + g * 16, 16)],
                               s3v)
                plsc.addupdate(acc_v.at[s_first, pl.ds(3 * _D + g * 16, 16)],
                               s4v)
            plsc.addupdate(acc_v.at[s_first, pl.ds(4 * _D, 16)],
                           ones16 * float(_G))

        return carry

    def body_slow(gi, carry):
        i0 = gi * _G
        idxv = idx_v[pl.ds(i0, _G)]
        s_first = idxv[0]
        s_last = idxv[_G - 1]
        ib = i0 * _D

        @pl.when(s_first != s_last)
        def _():
            for j in range(_G):
                s = idxv[j]
                jb = ib + j * _D
                for g in range(_D // 16):
                    x = chunk_v[pl.ds(jb + g * 16, 16)]
                    x2 = x * x
                    plsc.addupdate(acc_v.at[s, pl.ds(g * 16, 16)], x)
                    plsc.addupdate(acc_v.at[s, pl.ds(_D + g * 16, 16)], x2)
                    plsc.addupdate(acc_v.at[s, pl.ds(2 * _D + g * 16, 16)],
                                   x2 * x)
                    plsc.addupdate(acc_v.at[s, pl.ds(3 * _D + g * 16, 16)],
                                   x2 * x2)
                plsc.addupdate(acc_v.at[s, pl.ds(4 * _D, 16)], ones16)

        return carry

    lax.fori_loop(0, _NGROUPS, body_fast, 0)
    lax.fori_loop(0, _NGROUPS, body_slow, 0)

    plsc.subcore_barrier()

    @pl.when(narrow)
    def _():
        widx_v[pl.ds(0, 16)] = jnp.where(smin + iota16 <= smax,
                                         smin + iota16, _B)
        pltpu.sync_copy(acc_v.at[pl.ds(smin, _G)], shared.at[widx_v],
                        add=True)

    @pl.when(jnp.logical_not(narrow))
    def _():
        for k in range(_B // 16):
            fidx_v[pl.ds(k * 16, 16)] = iota16 + (k * 16)
        pltpu.sync_copy(acc_v.at[pl.ds(0, _B)], shared.at[fidx_v], add=True)

    plsc.subcore_barrier()

    pltpu.sync_copy(shared.at[pl.ds(4 * sid, 4)],
                    part_hbm.at[cid, pl.ds(4 * sid, 4)])


def _tc_moments(graph_ref, idx_ref, out_ref):
    x = graph_ref[...][:_TC_ROWS, :]
    ids = idx_ref[...][:, :_TC_ROWS]
    oh = (lax.broadcasted_iota(jnp.int32, (_B, _TC_ROWS), 0)
          == ids).astype(jnp.float32)
    x2 = x * x
    s1 = jnp.dot(oh, x, preferred_element_type=jnp.float32)
    s2 = jnp.dot(oh, x2, preferred_element_type=jnp.float32)
    s3 = jnp.dot(oh, x2 * x, preferred_element_type=jnp.float32)
    s4 = jnp.dot(oh, x2 * x2, preferred_element_type=jnp.float32)
    cnt = jnp.sum(oh, axis=1, keepdims=True)
    out_ref[...] = jnp.concatenate(
        [s1, s2, s3, s4, jnp.broadcast_to(cnt, (_B, 16)),
         jnp.zeros((_B, _ROW - 4 * _D - 16), jnp.float32)], axis=1)


def _finalize(part_ref, tcp_ref, out_ref):
    tot = jnp.sum(part_ref[...], axis=0) + tcp_ref[...]
    s1 = tot[:, 0:_D]
    s2 = tot[:, _D:2 * _D]
    s3 = tot[:, 2 * _D:3 * _D]
    s4 = tot[:, 3 * _D:4 * _D]
    cnt = tot[:, 4 * _D:4 * _D + 1]
    m = s1 / cnt
    m2 = s2 / cnt
    m3 = s3 / cnt
    m4 = s4 / cnt
    mm = m * m
    var = m2 - mm
    skew = m3 - 3.0 * m * m2 + 2.0 * m * mm
    kurt = m4 - 4.0 * m * m3 + 6.0 * mm * m2 - 3.0 * mm * mm - 3.0
    inf_val = 1000000000000000.0
    skew = jnp.where(skew > inf_val, 0.0, skew)
    skew = jnp.where(jnp.isnan(skew), 0.0, skew)
    kurt = jnp.where(kurt > inf_val, -3.0, kurt)
    kurt = jnp.where(jnp.isnan(kurt), -3.0, kurt)
    out_ref[...] = jnp.concatenate([m, var, skew, kurt], axis=1)


def kernel(graph, batch_indices):
    graph1d = jnp.reshape(graph, (-1,))
    part = pl.kernel(
        _sc_moments,
        out_type=jax.ShapeDtypeStruct((2, _B, _ROW), jnp.float32),
        mesh=plsc.VectorSubcoreMesh(core_axis_name="c", subcore_axis_name="s"),
        compiler_params=pltpu.CompilerParams(use_tc_tiling_on_sc=False),
        scratch_types=[
            pltpu.VMEM((_CHUNK * _D,), jnp.float32),
            pltpu.VMEM((_CHUNK + 16,), jnp.int32),
            pltpu.VMEM((_AROWS, _ROW), jnp.float32),
            pltpu.VMEM((5, _ROW), jnp.float32),
            pltpu.VMEM((16,), jnp.int32),
            pltpu.VMEM((_B,), jnp.int32),
            pltpu.VMEM_SHARED((5 * 16, _ROW), jnp.float32),
            pltpu.SemaphoreType.DMA,
            pltpu.SemaphoreType.DMA,
        ],
    )(graph1d, batch_indices)
    tc_part = pl.pallas_call(
        _tc_moments,
        out_shape=jax.ShapeDtypeStruct((_B, _ROW), jnp.float32),
    )(graph, jnp.reshape(batch_indices, (1, _N)))
    return pl.pallas_call(
        _finalize,
        out_shape=jax.ShapeDtypeStruct((_B, 4 * _D), jnp.float32),
    )(part, tc_part)


# speedup vs baseline: 1.3444x; 1.0023x over previous
import jax
import jax.numpy as jnp
from jax import lax
from jax.experimental import pallas as pl
from jax.experimental.pallas import tpu as pltpu
from jax.experimental.pallas import tpu_sc as plsc

_N = 10000
_D = 128
_B = 64
_ROW = 640
_NW = 32
_TC_ROWS = 4880
_CHUNK = (_N - _TC_ROWS) // _NW
_G = 16
_NGROUPS = _CHUNK // _G
_AROWS = _B + _G


def _sc_moments(graph_hbm, idx_hbm, part_hbm, chunk_v, idx_v, acc_v,
                zstg_v, widx_v, fidx_v, shared, gsem, isem):
    cid = lax.axis_index("c")
    sid = lax.axis_index("s")
    wid = sid * 2 + cid
    base = _TC_ROWS + wid * _CHUNK

    zeros16 = jnp.zeros((16,), jnp.float32)
    iota16 = lax.iota(jnp.int32, 16)

    gcopy = pltpu.async_copy(graph_hbm.at[pl.ds(base * _D, _CHUNK * _D)],
                             chunk_v, gsem)
    icopy = pltpu.async_copy(idx_hbm.at[pl.ds(base, _CHUNK)],
                             idx_v.at[pl.ds(0, _CHUNK)], isem)

    for k in range(5):
        for j in range(_ROW // 16):
            zstg_v[k, pl.ds(j * 16, 16)] = zeros16
    pltpu.sync_copy(zstg_v, shared.at[pl.ds(5 * sid, 5)])

    icopy.wait()
    smin = idx_v[pl.ds(0, 16)][0]
    smax = idx_v[pl.ds(_CHUNK - 16, 16)][15]
    narrow = (smax - smin) < _G

    @pl.when(narrow)
    def _():
        def zw(r, carry):
            for j in range(_ROW // 16):
                acc_v[r, pl.ds(j * 16, 16)] = zeros16
            return carry
        lax.fori_loop(smin, smin + _G, zw, 0)

    @pl.when(jnp.logical_not(narrow))
    def _():
        def zf(r, carry):
            for j in range(_ROW // 16):
                acc_v[r, pl.ds(j * 16, 16)] = zeros16
            return carry
        lax.fori_loop(0, _B, zf, 0)

    ones16 = jnp.ones((16,), jnp.float32)
    gcopy.wait()

    def body_fast(gi, carry):
        i0 = gi * _G
        idxv = idx_v[pl.ds(i0, _G)]
        s_first = idxv[0]
        s_last = idxv[_G - 1]
        ib = i0 * _D

        @pl.when(s_first == s_last)
        def _():
            for g in range(_D // 16):
                a1 = [zeros16] * 4
                a2 = [zeros16] * 4
                a3 = [zeros16] * 4
                a4 = [zeros16] * 4
                for j in range(_G):
                    w = j % 4
                    x = chunk_v[pl.ds(ib + j * _D + g * 16, 16)]
                    x2 = x * x
                    a1[w] = a1[w] + x
                    a2[w] = a2[w] + x2
                    a3[w] = a3[w] + x2 * x
                    a4[w] = a4[w] + x2 * x2
                s1v = (a1[0] + a1[1]) + (a1[2] + a1[3])
                s2v = (a2[0] + a2[1]) + (a2[2] + a2[3])
                s3v = (a3[0] + a3[1]) + (a3[2] + a3[3])
                s4v = (a4[0] + a4[1]) + (a4[2] + a4[3])
                plsc.addupdate(acc_v.at[s_first, pl.ds(g * 16, 16)], s1v)
                plsc.addupdate(acc_v.at[s_first, pl.ds(_D + g * 16, 16)], s2v)
                plsc.addupdate(acc_v.at[s_first, pl.ds(2 * _D + g * 16, 16)],
                               s3v)
                plsc.addupdate(acc_v.at[s_first, pl.ds(3 * _D + g * 16, 16)],
                               s4v)
            plsc.addupdate(acc_v.at[s_first, pl.ds(4 * _D, 16)],
                           ones16 * float(_G))

        return carry

    def body_slow(gi, carry):
        i0 = gi * _G
        idxv = idx_v[pl.ds(i0, _G)]
        s_first = idxv[0]
        s_last = idxv[_G - 1]
        ib = i0 * _D

        @pl.when(s_first != s_last)
        def _():
            for j in range(_G):
                s = idxv[j]
                jb = ib + j * _D
                for g in range(_D // 16):
                    x = chunk_v[pl.ds(jb + g * 16, 16)]
                    x2 = x * x
                    plsc.addupdate(acc_v.at[s, pl.ds(g * 16, 16)], x)
                    plsc.addupdate(acc_v.at[s, pl.ds(_D + g * 16, 16)], x2)
                    plsc.addupdate(acc_v.at[s, pl.ds(2 * _D + g * 16, 16)],
                                   x2 * x)
                    plsc.addupdate(acc_v.at[s, pl.ds(3 * _D + g * 16, 16)],
                                   x2 * x2)
                plsc.addupdate(acc_v.at[s, pl.ds(4 * _D, 16)], ones16)

        return carry

    lax.fori_loop(0, _NGROUPS, body_fast, 0)
    lax.fori_loop(0, _NGROUPS, body_slow, 0)

    plsc.subcore_barrier()

    @pl.when(narrow)
    def _():
        widx_v[pl.ds(0, 16)] = jnp.where(smin + iota16 <= smax,
                                         smin + iota16, _B)
        pltpu.sync_copy(acc_v.at[pl.ds(smin, _G)], shared.at[widx_v],
                        add=True)

    @pl.when(jnp.logical_not(narrow))
    def _():
        for k in range(_B // 16):
            fidx_v[pl.ds(k * 16, 16)] = iota16 + (k * 16)
        pltpu.sync_copy(acc_v.at[pl.ds(0, _B)], shared.at[fidx_v], add=True)

    plsc.subcore_barrier()

    pltpu.sync_copy(shared.at[pl.ds(4 * sid, 4)],
                    part_hbm.at[cid, pl.ds(4 * sid, 4)])


def _tc_moments(graph_ref, idx_ref, out_ref):
    x = graph_ref[...][:_TC_ROWS, :]
    ids = idx_ref[...][:, :_TC_ROWS]
    oh = (lax.broadcasted_iota(jnp.int32, (_B, _TC_ROWS), 0)
          == ids).astype(jnp.float32)
    x2 = x * x
    s1 = jnp.dot(oh, x, preferred_element_type=jnp.float32)
    s2 = jnp.dot(oh, x2, preferred_element_type=jnp.float32)
    s3 = jnp.dot(oh, x2 * x, preferred_element_type=jnp.float32)
    s4 = jnp.dot(oh, x2 * x2, preferred_element_type=jnp.float32)
    cnt = jnp.sum(oh, axis=1, keepdims=True)
    out_ref[...] = jnp.concatenate(
        [s1, s2, s3, s4, jnp.broadcast_to(cnt, (_B, 16)),
         jnp.zeros((_B, _ROW - 4 * _D - 16), jnp.float32)], axis=1)


def _finalize(part_ref, tcp_ref, out_ref):
    tot = jnp.sum(part_ref[...], axis=0) + tcp_ref[...]
    s1 = tot[:, 0:_D]
    s2 = tot[:, _D:2 * _D]
    s3 = tot[:, 2 * _D:3 * _D]
    s4 = tot[:, 3 * _D:4 * _D]
    cnt = tot[:, 4 * _D:4 * _D + 1]
    m = s1 / cnt
    m2 = s2 / cnt
    m3 = s3 / cnt
    m4 = s4 / cnt
    mm = m * m
    var = m2 - mm
    skew = m3 - 3.0 * m * m2 + 2.0 * m * mm
    kurt = m4 - 4.0 * m * m3 + 6.0 * mm * m2 - 3.0 * mm * mm - 3.0
    inf_val = 1000000000000000.0
    skew = jnp.where(skew > inf_val, 0.0, skew)
    skew = jnp.where(jnp.isnan(skew), 0.0, skew)
    kurt = jnp.where(kurt > inf_val, -3.0, kurt)
    kurt = jnp.where(jnp.isnan(kurt), -3.0, kurt)
    out_ref[...] = jnp.concatenate([m, var, skew, kurt], axis=1)


def kernel(graph, batch_indices):
    graph1d = jnp.reshape(graph, (-1,))
    part = pl.kernel(
        _sc_moments,
        out_type=jax.ShapeDtypeStruct((2, _B, _ROW), jnp.float32),
        mesh=plsc.VectorSubcoreMesh(core_axis_name="c", subcore_axis_name="s"),
        compiler_params=pltpu.CompilerParams(use_tc_tiling_on_sc=False,
                                             skip_device_barrier=True),
        scratch_types=[
            pltpu.VMEM((_CHUNK * _D,), jnp.float32),
            pltpu.VMEM((_CHUNK + 16,), jnp.int32),
            pltpu.VMEM((_AROWS, _ROW), jnp.float32),
            pltpu.VMEM((5, _ROW), jnp.float32),
            pltpu.VMEM((16,), jnp.int32),
            pltpu.VMEM((_B,), jnp.int32),
            pltpu.VMEM_SHARED((5 * 16, _ROW), jnp.float32),
            pltpu.SemaphoreType.DMA,
            pltpu.SemaphoreType.DMA,
        ],
    )(graph1d, batch_indices)
    tc_part = pl.pallas_call(
        _tc_moments,
        out_shape=jax.ShapeDtypeStruct((_B, _ROW), jnp.float32),
    )(graph, jnp.reshape(batch_indices, (1, _N)))
    return pl.pallas_call(
        _finalize,
        out_shape=jax.ShapeDtypeStruct((_B, 4 * _D), jnp.float32),
    )(part, tc_part)
